# Initial kernel scaffold; baseline (speedup 1.0000x reference)
import jax
import jax.numpy as jnp
from jax.experimental import pallas as pl


def kernel(init_features, W_init, b_init, W_agg, W_self, b_gnn, W1, b1, W2, b2, W3, b3, node_type, edge_index, out_idx):
    raise NotImplementedError("write your pallas kernel here")


# speedup vs baseline: 25.4130x; 25.4130x over previous
import functools

import jax
import jax.numpy as jnp
from jax import lax
from jax.experimental import pallas as pl
from jax.experimental.pallas import tpu as pltpu
from jax.experimental.pallas import tpu_sc as plsc

N = 10000
E = 320000
H = 128
K = 2048

NC = 2
NS = 16
NW = NC * NS
EPW = E // NW
NT3 = 3 * N
L = 16


def _sc_counts_kernel(nt_hbm, src_hbm, dst_hbm, oi_hbm, out_hbm,
                      nt_v, src_v, dst_v, cnt_v, oi_v, res_v):
    cid = lax.axis_index("c")
    sid = lax.axis_index("s")
    wid = sid * NC + cid

    pltpu.sync_copy(nt_hbm, nt_v)
    pltpu.sync_copy(src_hbm.at[pl.ds(wid * EPW, EPW)], src_v)
    pltpu.sync_copy(dst_hbm.at[pl.ds(wid * EPW, EPW)], dst_v)
    pltpu.sync_copy(oi_hbm, oi_v)

    zeros = jnp.zeros((L,), jnp.float32)
    ones = jnp.ones((L,), jnp.float32)

    def zero_body(i, carry):
        cnt_v[pl.ds(i * L, L)] = zeros
        return carry

    lax.fori_loop(0, NT3 // L, zero_body, 0)

    def edge_body(i, carry):
        s = src_v[pl.ds(i * L, L)]
        d = dst_v[pl.ds(i * L, L)]
        t = plsc.load_gather(nt_v, [s])
        addr = d * 3 + t
        runcnt, last = plsc.scan_count(addr)
        plsc.addupdate_scatter(
            cnt_v, [addr], runcnt.astype(jnp.float32), mask=last)
        return carry

    lax.fori_loop(0, EPW // L, edge_body, 0)

    marker = jnp.where(wid == 0, 1.0, 0.0)

    def read_body(j, carry):
        oi = oi_v[pl.ds(j * L, L)]
        t = plsc.load_gather(nt_v, [oi])
        base3 = oi * 3
        c0 = plsc.load_gather(cnt_v, [base3])
        c1 = plsc.load_gather(cnt_v, [base3 + 1])
        c2 = plsc.load_gather(cnt_v, [base3 + 2])
        res_v[pl.ds(0 * K + j * L, L)] = c0
        res_v[pl.ds(1 * K + j * L, L)] = c1
        res_v[pl.ds(2 * K + j * L, L)] = c2
        res_v[pl.ds(3 * K + j * L, L)] = t.astype(jnp.float32) * marker
        return carry

    lax.fori_loop(0, K // L, read_body, 0)

    pltpu.sync_copy(res_v, out_hbm.at[wid])


def _sc_counts(node_type, src, dst, out_idx):
    mesh = plsc.VectorSubcoreMesh(core_axis_name="c", subcore_axis_name="s")
    kern = functools.partial(
        pl.kernel,
        mesh=mesh,
        compiler_params=pltpu.CompilerParams(needs_layout_passes=False),
        out_type=jax.ShapeDtypeStruct((NW, 4 * K), jnp.float32),
        scratch_types=[
            pltpu.VMEM((N,), jnp.int32),
            pltpu.VMEM((EPW,), jnp.int32),
            pltpu.VMEM((EPW,), jnp.int32),
            pltpu.VMEM((NT3,), jnp.float32),
            pltpu.VMEM((K,), jnp.int32),
            pltpu.VMEM((4 * K,), jnp.float32),
        ],
    )(_sc_counts_kernel)
    return kern(node_type, src, dst, out_idx)


def _tc_mlp_kernel(part_ref, initf_ref, winit_ref, binit_ref, wagg_ref,
                   wself_ref, bgnn_ref, w1_ref, b1_ref, w2_ref, b2_ref,
                   w3_ref, b3_ref, out_ref):
    psum = jnp.sum(part_ref[...], axis=0)
    c3 = psum[0:3, :]
    tb = psum[3:4, :]

    contract0 = (((0,), (0,)), ((), ()))
    contract1 = (((1,), (0,)), ((), ()))
    hp = jax.lax.Precision.HIGHEST
    bf = jnp.bfloat16
    f32 = jnp.float32

    def dotT(a, b):
        return lax.dot_general(a, b, contract0, precision=hp)

    def dot_bf16(a, b, dims):
        return lax.dot_general(a.astype(bf), b.astype(bf), dims,
                               preferred_element_type=f32)

    initf = initf_ref[...]
    winit = winit_ref[...]
    ce = jnp.concatenate(
        [dot_bf16(initf[t:t + 1, :], winit[t], contract1)
         for t in range(3)], axis=0
    ) + binit_ref[...]

    m3 = dot_bf16(ce, wagg_ref[...], contract1)
    s3 = dot_bf16(ce, wself_ref[...], contract1)

    xt = dotT(m3, c3)

    onehot = jnp.concatenate(
        [(tb == float(t)).astype(jnp.float32) for t in range(3)], axis=0
    )
    st = dotT(s3, onehot)

    ht = jnp.maximum(xt + st + bgnn_ref[...], 0.0)
    h1 = jnp.maximum(dot_bf16(w1_ref[...], ht, contract0) + b1_ref[...], 0.0)
    h2 = jnp.maximum(dot_bf16(w2_ref[...], h1, contract0) + b2_ref[...], 0.0)
    o = dot_bf16(w3_ref[...], h2, contract0) + b3_ref[...]
    out_ref[...] = 1.0 / (1.0 + jnp.exp(-o))


def _tc_mlp(partials, init_features, W_init, b_init, W_agg, W_self, b_gnn,
            W1, b1, W2, b2, W3, b3):
    return pl.pallas_call(
        _tc_mlp_kernel,
        out_shape=jax.ShapeDtypeStruct((1, K), jnp.float32),
    )(partials, init_features, W_init, b_init, W_agg, W_self, b_gnn,
      W1, b1, W2, b2, W3, b3)


@jax.jit
def kernel(init_features, W_init, b_init, W_agg, W_self, b_gnn,
           W1, b1, W2, b2, W3, b3, node_type, edge_index, out_idx):
    src = edge_index[0]
    dst = edge_index[1]
    partials = _sc_counts(node_type, src, dst, out_idx)
    partials = partials.reshape(NW, 4, K)
    o = _tc_mlp(partials, init_features, W_init, b_init, W_agg, W_self,
                b_gnn.reshape(H, 1), W1, b1.reshape(H, 1), W2,
                b2.reshape(H, 1), W3, b3.reshape(1, 1))
    return o.reshape(K)


# speedup vs baseline: 36.0433x; 1.4183x over previous
import functools

import jax
import jax.numpy as jnp
from jax import lax
from jax.experimental import pallas as pl
from jax.experimental.pallas import tpu as pltpu
from jax.experimental.pallas import tpu_sc as plsc

N = 10000
E = 320000
H = 128
K = 2048

NC = 2
NS = 16
NW = NC * NS
EPW = E // NW
NT3 = 3 * N
L = 16


ZU = 8
EU = 4


def _sc_counts_kernel(nt_hbm, ei_hbm, oi_hbm, out_hbm,
                      nt_v, src_v, dst_v, cnt_v, oi_v, res_v):
    cid = lax.axis_index("c")
    sid = lax.axis_index("s")
    wid = sid * NC + cid

    pltpu.sync_copy(nt_hbm, nt_v)
    pltpu.sync_copy(ei_hbm.at[pl.ds(wid * EPW, EPW)], src_v)
    pltpu.sync_copy(ei_hbm.at[pl.ds(E + wid * EPW, EPW)], dst_v)
    pltpu.sync_copy(oi_hbm, oi_v)

    zeros = jnp.zeros((L,), jnp.float32)

    def zero_body(i, carry):
        for u in range(ZU):
            cnt_v[pl.ds((i * ZU + u) * L, L)] = zeros
        return carry

    lax.fori_loop(0, NT3 // (L * ZU), zero_body, 0)

    def edge_step(i):
        s = src_v[pl.ds(i * L, L)]
        d = dst_v[pl.ds(i * L, L)]
        t = plsc.load_gather(nt_v, [s])
        addr = d * 3 + t
        runcnt, last = plsc.scan_count(addr)
        plsc.addupdate_scatter(
            cnt_v, [addr], runcnt.astype(jnp.float32), mask=last)

    def edge_body(i, carry):
        for u in range(EU):
            edge_step(i * EU + u)
        return carry

    lax.fori_loop(0, EPW // (L * EU), edge_body, 0)

    marker = jnp.where(wid == 0, 1.0, 0.0)

    def read_body(j, carry):
        oi = oi_v[pl.ds(j * L, L)]
        t = plsc.load_gather(nt_v, [oi])
        base3 = oi * 3
        c0 = plsc.load_gather(cnt_v, [base3])
        c1 = plsc.load_gather(cnt_v, [base3 + 1])
        c2 = plsc.load_gather(cnt_v, [base3 + 2])
        res_v[0, pl.ds(j * L, L)] = c0
        res_v[1, pl.ds(j * L, L)] = c1
        res_v[2, pl.ds(j * L, L)] = c2
        res_v[3, pl.ds(j * L, L)] = t.astype(jnp.float32) * marker
        return carry

    lax.fori_loop(0, K // L, read_body, 0)

    pltpu.sync_copy(res_v, out_hbm.at[wid])


def _sc_counts(node_type, edge_index, out_idx):
    mesh = plsc.VectorSubcoreMesh(core_axis_name="c", subcore_axis_name="s")
    kern = functools.partial(
        pl.kernel,
        mesh=mesh,
        compiler_params=pltpu.CompilerParams(needs_layout_passes=False),
        out_type=jax.ShapeDtypeStruct((NW, 4, K), jnp.float32),
        scratch_types=[
            pltpu.VMEM((N,), jnp.int32),
            pltpu.VMEM((EPW,), jnp.int32),
            pltpu.VMEM((EPW,), jnp.int32),
            pltpu.VMEM((NT3,), jnp.float32),
            pltpu.VMEM((K,), jnp.int32),
            pltpu.VMEM((4, K), jnp.float32),
        ],
    )(_sc_counts_kernel)
    return kern(node_type, edge_index, out_idx)


def _tc_mlp_kernel(part_ref, initf_ref, winit_ref, binit_ref, wagg_ref,
                   wself_ref, bgnn_ref, w1_ref, b1_ref, w2_ref, b2_ref,
                   w3_ref, b3_ref, out_ref):
    psum = jnp.sum(part_ref[...], axis=0)
    c3 = psum[0:3, :]
    tb = psum[3:4, :]

    contract0 = (((0,), (0,)), ((), ()))
    contract1 = (((1,), (0,)), ((), ()))
    hp = jax.lax.Precision.HIGHEST
    bf = jnp.bfloat16
    f32 = jnp.float32

    def dotT(a, b):
        return lax.dot_general(a, b, contract0, precision=hp)

    def dot_bf16(a, b, dims):
        return lax.dot_general(a.astype(bf), b.astype(bf), dims,
                               preferred_element_type=f32)

    initf = initf_ref[...]
    winit = winit_ref[...]
    ce = jnp.concatenate(
        [dot_bf16(initf[t:t + 1, :], winit[t], contract1)
         for t in range(3)], axis=0
    ) + binit_ref[...]

    m3 = dot_bf16(ce, wagg_ref[...], contract1)
    s3 = dot_bf16(ce, wself_ref[...], contract1)

    xt = dotT(m3, c3)

    onehot = jnp.concatenate(
        [(tb == float(t)).astype(jnp.float32) for t in range(3)], axis=0
    )
    st = dotT(s3, onehot)

    ht = jnp.maximum(xt + st + bgnn_ref[...], 0.0)
    h1 = jnp.maximum(dot_bf16(w1_ref[...], ht, contract0) + b1_ref[...], 0.0)
    h2 = jnp.maximum(dot_bf16(w2_ref[...], h1, contract0) + b2_ref[...], 0.0)
    o = dot_bf16(w3_ref[...], h2, contract0) + b3_ref[...]
    out_ref[...] = 1.0 / (1.0 + jnp.exp(-o))


def _tc_mlp(partials, init_features, W_init, b_init, W_agg, W_self, b_gnn,
            W1, b1, W2, b2, W3, b3):
    return pl.pallas_call(
        _tc_mlp_kernel,
        out_shape=jax.ShapeDtypeStruct((1, K), jnp.float32),
    )(partials, init_features, W_init, b_init, W_agg, W_self, b_gnn,
      W1, b1, W2, b2, W3, b3)


@jax.jit
def kernel(init_features, W_init, b_init, W_agg, W_self, b_gnn,
           W1, b1, W2, b2, W3, b3, node_type, edge_index, out_idx):
    partials = _sc_counts(node_type, edge_index.reshape(2 * E), out_idx)
    o = _tc_mlp(partials, init_features, W_init, b_init, W_agg, W_self,
                b_gnn.reshape(H, 1), W1, b1.reshape(H, 1), W2,
                b2.reshape(H, 1), W3, b3.reshape(1, 1))
    return o.reshape(K)


# speedup vs baseline: 43.1140x; 1.1962x over previous
import functools

import jax
import jax.numpy as jnp
from jax import lax
from jax.experimental import pallas as pl
from jax.experimental.pallas import tpu as pltpu
from jax.experimental.pallas import tpu_sc as plsc

N = 10000
E = 320000
H = 128
K = 2048

NC = 2
NS = 16
NW = NC * NS
EPW = E // NW
NT3 = 3 * N
L = 16


ZU = 8
EU = 8


def _sc_counts_kernel(nt_hbm, ei_hbm, oi_hbm, out_hbm,
                      nt_v, src_v, dst_v, cnt_v, oi_v, res_v,
                      sem0, sem1, sem2, sem3):
    cid = lax.axis_index("c")
    sid = lax.axis_index("s")
    wid = sid * NC + cid

    c0 = pltpu.async_copy(nt_hbm, nt_v, sem0)
    c1 = pltpu.async_copy(ei_hbm.at[pl.ds(wid * EPW, EPW)], src_v, sem1)
    c2 = pltpu.async_copy(ei_hbm.at[pl.ds(E + wid * EPW, EPW)], dst_v, sem2)
    c3 = pltpu.async_copy(oi_hbm, oi_v, sem3)

    zeros = jnp.zeros((L,), jnp.float32)
    ones = jnp.ones((L,), jnp.float32)

    def zero_body(i, carry):
        for u in range(ZU):
            cnt_v[pl.ds((i * ZU + u) * L, L)] = zeros
        return carry

    lax.fori_loop(0, NT3 // (L * ZU), zero_body, 0)
    c0.wait(); c1.wait(); c2.wait(); c3.wait()

    def edge_step(i):
        s = src_v[pl.ds(i * L, L)]
        d = dst_v[pl.ds(i * L, L)]
        t = plsc.load_gather(nt_v, [s])
        addr = d * 3 + t
        plsc.addupdate_scatter(cnt_v, [addr], ones)

    def edge_body(i, carry):
        for u in range(EU):
            edge_step(i * EU + u)
        return carry

    lax.fori_loop(0, EPW // (L * EU), edge_body, 0)

    marker = jnp.where(wid == 0, 1.0, 0.0)

    def read_body(j, carry):
        oi = oi_v[pl.ds(j * L, L)]
        t = plsc.load_gather(nt_v, [oi])
        base3 = oi * 3
        c0 = plsc.load_gather(cnt_v, [base3])
        c1 = plsc.load_gather(cnt_v, [base3 + 1])
        c2 = plsc.load_gather(cnt_v, [base3 + 2])
        res_v[0, pl.ds(j * L, L)] = c0
        res_v[1, pl.ds(j * L, L)] = c1
        res_v[2, pl.ds(j * L, L)] = c2
        res_v[3, pl.ds(j * L, L)] = t.astype(jnp.float32) * marker
        return carry

    lax.fori_loop(0, K // L, read_body, 0)

    pltpu.sync_copy(res_v, out_hbm.at[wid])


def _sc_counts(node_type, edge_index, out_idx):
    mesh = plsc.VectorSubcoreMesh(core_axis_name="c", subcore_axis_name="s")
    kern = functools.partial(
        pl.kernel,
        mesh=mesh,
        compiler_params=pltpu.CompilerParams(needs_layout_passes=False),
        out_type=jax.ShapeDtypeStruct((NW, 4, K), jnp.float32),
        scratch_types=[
            pltpu.VMEM((N,), jnp.int32),
            pltpu.VMEM((EPW,), jnp.int32),
            pltpu.VMEM((EPW,), jnp.int32),
            pltpu.VMEM((NT3,), jnp.float32),
            pltpu.VMEM((K,), jnp.int32),
            pltpu.VMEM((4, K), jnp.float32),
            pltpu.SemaphoreType.DMA,
            pltpu.SemaphoreType.DMA,
            pltpu.SemaphoreType.DMA,
            pltpu.SemaphoreType.DMA,
        ],
    )(_sc_counts_kernel)
    return kern(node_type, edge_index, out_idx)


def _tc_mlp_kernel(part_ref, initf_ref, winit_ref, binit_ref, wagg_ref,
                   wself_ref, bgnn_ref, w1_ref, b1_ref, w2_ref, b2_ref,
                   w3_ref, b3_ref, out_ref):
    psum = jnp.sum(part_ref[...], axis=0)
    c3 = psum[0:3, :]
    tb = psum[3:4, :]

    contract0 = (((0,), (0,)), ((), ()))
    contract1 = (((1,), (0,)), ((), ()))
    hp = jax.lax.Precision.HIGHEST
    bf = jnp.bfloat16
    f32 = jnp.float32

    def dotT(a, b):
        return lax.dot_general(a, b, contract0, precision=hp)

    def dot_bf16(a, b, dims):
        return lax.dot_general(a.astype(bf), b.astype(bf), dims,
                               preferred_element_type=f32)

    initf = initf_ref[...]
    winit = winit_ref[...]
    ce = jnp.concatenate(
        [dot_bf16(initf[t:t + 1, :], winit[t], contract1)
         for t in range(3)], axis=0
    ) + binit_ref[...]

    m3 = dot_bf16(ce, wagg_ref[...], contract1)
    s3 = dot_bf16(ce, wself_ref[...], contract1)

    xt = dotT(m3, c3)

    onehot = jnp.concatenate(
        [(tb == float(t)).astype(jnp.float32) for t in range(3)], axis=0
    )
    st = dotT(s3, onehot)

    ht = jnp.maximum(xt + st + bgnn_ref[...], 0.0)
    h1 = jnp.maximum(dot_bf16(w1_ref[...], ht, contract0) + b1_ref[...], 0.0)
    h2 = jnp.maximum(dot_bf16(w2_ref[...], h1, contract0) + b2_ref[...], 0.0)
    o = dot_bf16(w3_ref[...], h2, contract0) + b3_ref[...]
    out_ref[...] = 1.0 / (1.0 + jnp.exp(-o))


def _tc_mlp(partials, init_features, W_init, b_init, W_agg, W_self, b_gnn,
            W1, b1, W2, b2, W3, b3):
    return pl.pallas_call(
        _tc_mlp_kernel,
        out_shape=jax.ShapeDtypeStruct((1, K), jnp.float32),
    )(partials, init_features, W_init, b_init, W_agg, W_self, b_gnn,
      W1, b1, W2, b2, W3, b3)


@jax.jit
def kernel(init_features, W_init, b_init, W_agg, W_self, b_gnn,
           W1, b1, W2, b2, W3, b3, node_type, edge_index, out_idx):
    partials = _sc_counts(node_type, edge_index.reshape(2 * E), out_idx)
    o = _tc_mlp(partials, init_features, W_init, b_init, W_agg, W_self,
                b_gnn.reshape(H, 1), W1, b1.reshape(H, 1), W2,
                b2.reshape(H, 1), W3, b3.reshape(1, 1))
    return o.reshape(K)
